# baseline (speedup 1.0000x reference)
import functools

import jax
import jax.numpy as jnp
from jax import lax
from jax.experimental import pallas as pl
from jax.experimental.pallas import tpu as pltpu
from jax.experimental.pallas import tpu_sc as plsc

B = 4
L_REMAIN = 2048
D = 1024
N = 8192
LFULL = N + 1
MASK_ROW = B * L_REMAIN
IW = 16
WR = B * IW
SQ = N // IW
NC, NS = 2, 16
NW = NC * NS
WPT = SQ // NW
PAD_ROW = WR


def _sc_revert(table, ridx_flat, pos_enc, pm_flat):
    mesh = plsc.VectorSubcoreMesh(core_axis_name="c", subcore_axis_name="s")

    @functools.partial(
        pl.kernel,
        out_type=jax.ShapeDtypeStruct((B * LFULL * D,), jnp.float32),
        mesh=mesh,
        compiler_params=pltpu.CompilerParams(needs_layout_passes=False),
        scratch_types=[
            pltpu.VMEM((B * L_REMAIN,), jnp.int32),
            pltpu.VMEM((B * 24,), jnp.int32),
            pltpu.VMEM((80,), jnp.int32),
            pltpu.VMEM((80,), jnp.int32),
            pltpu.VMEM((80,), jnp.int32),
            pltpu.VMEM((80,), jnp.int32),
            pltpu.VMEM((WR,), jnp.int32),
            pltpu.VMEM((WR,), jnp.int32),
            pltpu.VMEM((16, D), jnp.float32),
            pltpu.VMEM((16, D), jnp.float32),
            pltpu.VMEM((IW, D), jnp.float32),
            pltpu.VMEM(((WR + 1) * D,), jnp.float32),
            pltpu.SemaphoreType.DMA,
            pltpu.SemaphoreType.DMA,
            pltpu.SemaphoreType.DMA,
            pltpu.SemaphoreType.DMA,
        ],
    )
    def k(table_hbm, ridx_hbm, pos_hbm, pm_hbm, out_hbm,
          pm_v, rscr, cidx0, cidx1, cdst0, cdst1, val0, val1, ga0, ga1,
          pos_v, wbuf, rsem, gsem0, gsem1, psem):
        cidx_s = (cidx0, cidx1)
        cdst_s = (cdst0, cdst1)
        val_s = (val0, val1)
        ga_s = (ga0, ga1)
        gsem_s = (gsem0, gsem1)

        wid = lax.axis_index("s") * NC + lax.axis_index("c")
        pltpu.sync_copy(pm_hbm, pm_v)
        lanes = lax.iota(jnp.int32, 16)

        def win_i0(n):
            return pl.multiple_of((wid * WPT + n) * IW, IW)

        def issue_pos(n):
            pltpu.async_copy(pos_hbm.at[pl.ds(win_i0(n), IW), :], pos_v, psem)

        def wait_pos(n):
            pltpu.make_async_copy(pos_hbm.at[pl.ds(win_i0(n), IW), :], pos_v,
                                  psem).wait()

        def start(n, s):
            i0 = win_i0(n)
            a0 = pl.multiple_of(jnp.maximum(i0 - 8, 0), 8)
            d0 = i0 - a0
            for b in range(B):
                pltpu.async_copy(ridx_hbm.at[pl.ds(b * N + a0, 24)],
                                 rscr.at[pl.ds(24 * b, 24)], rsem)
            for b in range(B):
                pltpu.make_async_copy(ridx_hbm.at[pl.ds(b * N + a0, 24)],
                                      rscr.at[pl.ds(24 * b, 24)], rsem).wait()
            ivec = i0 + lanes
            srcs, masks, counts = [], [], []
            for b in range(B):
                scr_idx = b * 24 + jnp.clip(lanes + d0 - 1, 0, 23)
                r = plsc.load_gather(rscr, [scr_idx])
                j = r + 1
                in_rng = (ivec > 0) & (j <= L_REMAIN - 1)
                fp = jnp.clip(b * L_REMAIN + j - 1, 0, B * L_REMAIN - 1)
                pmv = plsc.load_gather(pm_v, [fp])
                mb = in_rng & (pmv == 1)
                srcs.append(jnp.where(mb, b * L_REMAIN + j, MASK_ROW))
                masks.append(mb)
                counts.append(jnp.sum(mb.astype(jnp.int32)))
            o = jnp.int32(0)
            for b in range(B):
                plsc.store_compressed(cidx_s[s].at[pl.ds(o, 16)], srcs[b],
                                      mask=masks[b])
                plsc.store_compressed(cdst_s[s].at[pl.ds(o, 16)],
                                      16 * b + lanes, mask=masks[b])
                val_s[s][pl.ds(16 * b, 16)] = masks[b].astype(jnp.int32)
                o = o + counts[b]
            v = o
            cidx_s[s][pl.ds(v, 16)] = jnp.full((16,), MASK_ROW, jnp.int32)
            cdst_s[s][pl.ds(v, 16)] = jnp.full((16,), PAD_ROW, jnp.int32)
            pltpu.async_copy(table_hbm.at[cidx_s[s].at[pl.ds(0, 16)]],
                             ga_s[s], gsem_s[s])
            return v

        def merge_group(s, grp):
            def mb(k2, carry2):
                e = grp * 16 + k2
                dstb = plsc.load_gather(cdst_s[s],
                                        [jnp.full((16,), e, jnp.int32)])
                base = dstb * D
                for kk in range(D // 16):
                    col = base + kk * 16 + lanes
                    plsc.store_scatter(wbuf, [col],
                                       ga_s[s][k2, pl.ds(kk * 16, 16)])
                return carry2
            lax.fori_loop(0, 16, mb, 0)

        def finish(n, s, v):
            i0 = win_i0(n)
            pltpu.make_async_copy(table_hbm.at[cidx_s[s].at[pl.ds(0, 16)]],
                                  ga_s[s], gsem_s[s]).wait()
            merge_group(s, 0)
            for grp in range(1, 5):
                @pl.when(v >= 16 * grp)
                def _(grp=grp):
                    pltpu.async_copy(
                        table_hbm.at[cidx_s[s].at[pl.ds(16 * grp, 16)]],
                        ga_s[s], gsem_s[s]).wait()
                    merge_group(s, grp)
            wait_pos(n)

            def ab(r, carry2):
                il = r - (r // IW) * IW
                vb = plsc.load_gather(val_s[s], [jnp.full((16,), r, jnp.int32)])
                base = r * D
                for kk in range(D // 16):
                    sl = pl.ds(kk * 16, 16)
                    w = wbuf[pl.ds(base + kk * 16, 16)]
                    mt = wbuf[pl.ds(PAD_ROW * D + kk * 16, 16)]
                    wbuf[pl.ds(base + kk * 16, 16)] = (
                        jnp.where(vb != 0, w, mt) + pos_v[il, sl])
                return carry2

            lax.fori_loop(0, WR, ab, 0)
            for b in range(B):
                pltpu.sync_copy(
                    wbuf.at[pl.ds(b * IW * D, IW * D)],
                    out_hbm.at[pl.ds((b * LFULL + i0) * D, IW * D)])

            @pl.when(n + 1 < WPT)
            def _():
                issue_pos(n + 1)

        issue_pos(0)
        va = start(0, 0)
        vb = start(1, 1)

        def body(g, carry):
            va, vb = carry
            finish(2 * g, 0, va)
            va2 = start(2 * g + 2, 0)
            finish(2 * g + 1, 1, vb)
            vb2 = start(2 * g + 3, 1)
            return (va2, vb2)

        va, vb = lax.fori_loop(0, WPT // 2 - 1, body, (va, vb))
        finish(WPT - 2, 0, va)
        finish(WPT - 1, 1, vb)

        @pl.when(wid < B)
        def _():
            bt = wid
            pltpu.sync_copy(ridx_hbm.at[pl.ds(bt * N + N - 16, 16)],
                            rscr.at[pl.ds(0, 16)])
            r = plsc.load_gather(rscr, [jnp.full((16,), 15, jnp.int32)])
            j = r + 1
            in_rng = j <= L_REMAIN - 1
            fp = jnp.clip(bt * L_REMAIN + j - 1, 0, B * L_REMAIN - 1)
            pmv = plsc.load_gather(pm_v, [fp])
            valid = in_rng & (pmv == 1)
            cidx0[pl.ds(0, 16)] = jnp.where(valid, bt * L_REMAIN + j, MASK_ROW)
            pltpu.async_copy(table_hbm.at[cidx0.at[pl.ds(0, 16)]],
                             ga0, gsem0).wait()
            pltpu.async_copy(pos_hbm.at[pl.ds(N, IW), :], pos_v, psem).wait()
            for kk in range(D // 16):
                sl = pl.ds(kk * 16, 16)
                wbuf[pl.ds(kk * 16, 16)] = ga0[0, sl] + pos_v[0, sl]
            pltpu.sync_copy(wbuf.at[pl.ds(0, D)],
                            out_hbm.at[pl.ds((bt * LFULL + N) * D, D)])

    return k(table, ridx_flat, pos_enc, pm_flat)


def kernel(temporal_data, revert_idx, temporal_pos_enc, remain_padding_mask, mask_token):
    table = jnp.concatenate(
        [temporal_data.reshape(B * L_REMAIN, D), mask_token], axis=0)
    ridx_flat = revert_idx.reshape(B * N)
    pm_flat = jnp.pad(remain_padding_mask, ((0, 0), (0, 1))).reshape(B * L_REMAIN)
    out = _sc_revert(table, ridx_flat, temporal_pos_enc, pm_flat)
    return out.reshape(B, LFULL, D)


# speedup vs baseline: 1.7884x; 1.7884x over previous
import functools

import jax
import jax.numpy as jnp
from jax import lax
from jax.experimental import pallas as pl
from jax.experimental.pallas import tpu as pltpu
from jax.experimental.pallas import tpu_sc as plsc

B = 4
L_REMAIN = 2048
D = 1024
N = 8192
LFULL = N + 1
MASK_ROW = B * L_REMAIN
IW = 16
WR = B * IW
SQ = N // IW
NC, NS = 2, 16
NW = NC * NS
WPT = SQ // NW


def _sc_revert(table, ridx_flat, pos_enc, pm_flat):
    mesh = plsc.VectorSubcoreMesh(core_axis_name="c", subcore_axis_name="s")

    @functools.partial(
        pl.kernel,
        out_type=jax.ShapeDtypeStruct((B, LFULL, D), jnp.float32),
        mesh=mesh,
        compiler_params=pltpu.CompilerParams(needs_layout_passes=False),
        scratch_types=[
            pltpu.VMEM((B * L_REMAIN,), jnp.int32),
            pltpu.VMEM((B * 24,), jnp.int32),
            pltpu.VMEM((80,), jnp.int32),
            pltpu.VMEM((80,), jnp.int32),
            pltpu.VMEM((80,), jnp.int32),
            pltpu.VMEM((80,), jnp.int32),
            pltpu.VMEM((WR,), jnp.int32),
            pltpu.VMEM((WR,), jnp.int32),
            pltpu.VMEM((16, D), jnp.float32),
            pltpu.VMEM((16, D), jnp.float32),
            pltpu.VMEM((IW, D), jnp.float32),
            pltpu.VMEM((WR, D), jnp.float32),
            pltpu.VMEM((D,), jnp.float32),
            pltpu.SemaphoreType.DMA,
            pltpu.SemaphoreType.DMA,
            pltpu.SemaphoreType.DMA,
            pltpu.SemaphoreType.DMA,
        ],
    )
    def k(table_hbm, ridx_hbm, pos_hbm, pm_hbm, out_hbm,
          pm_v, rscr, cidx0, cidx1, cdst0, cdst1, val0, val1, ga0, ga1,
          pos_v, wbuf, mt_v, rsem, gsem0, gsem1, psem):
        cidx_s = (cidx0, cidx1)
        cdst_s = (cdst0, cdst1)
        val_s = (val0, val1)
        ga_s = (ga0, ga1)
        gsem_s = (gsem0, gsem1)

        wid = lax.axis_index("s") * NC + lax.axis_index("c")
        pltpu.sync_copy(pm_hbm, pm_v)
        lanes = lax.iota(jnp.int32, 16)
        cidx0[pl.ds(0, 16)] = jnp.full((16,), MASK_ROW, jnp.int32)
        pltpu.async_copy(table_hbm.at[cidx0.at[pl.ds(0, 16)]], ga0, gsem0).wait()
        for kk in range(D // 16):
            mt_v[pl.ds(kk * 16, 16)] = ga0[0, pl.ds(kk * 16, 16)]

        def win_i0(n):
            return pl.multiple_of((wid * WPT + n) * IW, IW)

        def issue_pos(n):
            pltpu.async_copy(pos_hbm.at[pl.ds(win_i0(n), IW), :], pos_v, psem)

        def wait_pos(n):
            pltpu.make_async_copy(pos_hbm.at[pl.ds(win_i0(n), IW), :], pos_v,
                                  psem).wait()

        def start(n, s):
            i0 = win_i0(n)
            a0 = pl.multiple_of(jnp.maximum(i0 - 8, 0), 8)
            d0 = i0 - a0
            for b in range(B):
                pltpu.async_copy(ridx_hbm.at[pl.ds(b * N + a0, 24)],
                                 rscr.at[pl.ds(24 * b, 24)], rsem)
            for b in range(B):
                pltpu.make_async_copy(ridx_hbm.at[pl.ds(b * N + a0, 24)],
                                      rscr.at[pl.ds(24 * b, 24)], rsem).wait()
            ivec = i0 + lanes
            srcs, masks, counts = [], [], []
            for b in range(B):
                scr_idx = b * 24 + jnp.clip(lanes + d0 - 1, 0, 23)
                r = plsc.load_gather(rscr, [scr_idx])
                j = r + 1
                in_rng = (ivec > 0) & (j <= L_REMAIN - 1)
                fp = jnp.clip(b * L_REMAIN + j - 1, 0, B * L_REMAIN - 1)
                pmv = plsc.load_gather(pm_v, [fp])
                mb = in_rng & (pmv == 1)
                srcs.append(jnp.where(mb, b * L_REMAIN + j, MASK_ROW))
                masks.append(mb)
                counts.append(jnp.sum(mb.astype(jnp.int32)))
            o = jnp.int32(0)
            for b in range(B):
                plsc.store_compressed(cidx_s[s].at[pl.ds(o, 16)], srcs[b],
                                      mask=masks[b])
                plsc.store_compressed(cdst_s[s].at[pl.ds(o, 16)],
                                      16 * b + lanes, mask=masks[b])
                val_s[s][pl.ds(16 * b, 16)] = masks[b].astype(jnp.int32)
                o = o + counts[b]
            v = o
            cidx_s[s][pl.ds(v, 16)] = jnp.full((16,), MASK_ROW, jnp.int32)
            cdst_s[s][pl.ds(v, 16)] = jnp.full((16,), WR - 1, jnp.int32)
            pltpu.async_copy(table_hbm.at[cidx_s[s].at[pl.ds(0, 16)]],
                             ga_s[s], gsem_s[s])
            return v

        def merge_group(s, grp, v):
            def mb(k2, carry2):
                e = grp * 16 + k2
                ev = jnp.full((16,), 1, jnp.int32) * e
                live = ev < v
                dstb = plsc.load_gather(cdst_s[s],
                                        [jnp.full((16,), e, jnp.int32)])
                dstb = jnp.minimum(dstb, WR - 1)
                for kk in range(D // 16):
                    col = kk * 16 + lanes
                    plsc.store_scatter(wbuf, [dstb, col],
                                       ga_s[s][k2, pl.ds(kk * 16, 16)],
                                       mask=live)
                return carry2
            lax.fori_loop(0, 16, mb, 0)

        def finish(n, s, v):
            i0 = win_i0(n)
            pltpu.make_async_copy(table_hbm.at[cidx_s[s].at[pl.ds(0, 16)]],
                                  ga_s[s], gsem_s[s]).wait()
            merge_group(s, 0, v)
            for grp in range(1, 5):
                @pl.when(v > 16 * grp)
                def _(grp=grp):
                    pltpu.async_copy(
                        table_hbm.at[cidx_s[s].at[pl.ds(16 * grp, 16)]],
                        ga_s[s], gsem_s[s]).wait()
                    merge_group(s, grp, v)
            wait_pos(n)

            def ab(r, carry2):
                il = r - (r // IW) * IW
                vb = plsc.load_gather(val_s[s], [jnp.full((16,), r, jnp.int32)])
                for kk in range(D // 16):
                    sl = pl.ds(kk * 16, 16)
                    w = wbuf[r, sl]
                    wbuf[r, sl] = (
                        jnp.where(vb != 0, w, mt_v[sl]) + pos_v[il, sl])
                return carry2

            lax.fori_loop(0, WR, ab, 0)
            for b in range(B):
                pltpu.sync_copy(
                    wbuf.at[pl.ds(b * IW, IW), :],
                    out_hbm.at[b, pl.ds(i0, IW), :])

            @pl.when(n + 1 < WPT)
            def _():
                issue_pos(n + 1)

        issue_pos(0)
        va = start(0, 0)
        vb = start(1, 1)

        def body(g, carry):
            va, vb = carry
            finish(2 * g, 0, va)
            va2 = start(2 * g + 2, 0)
            finish(2 * g + 1, 1, vb)
            vb2 = start(2 * g + 3, 1)
            return (va2, vb2)

        va, vb = lax.fori_loop(0, WPT // 2 - 1, body, (va, vb))
        finish(WPT - 2, 0, va)
        finish(WPT - 1, 1, vb)

        @pl.when(wid < B)
        def _():
            bt = wid
            pltpu.sync_copy(ridx_hbm.at[pl.ds(bt * N + N - 16, 16)],
                            rscr.at[pl.ds(0, 16)])
            r = plsc.load_gather(rscr, [jnp.full((16,), 15, jnp.int32)])
            j = r + 1
            in_rng = j <= L_REMAIN - 1
            fp = jnp.clip(bt * L_REMAIN + j - 1, 0, B * L_REMAIN - 1)
            pmv = plsc.load_gather(pm_v, [fp])
            valid = in_rng & (pmv == 1)
            cidx0[pl.ds(0, 16)] = jnp.where(valid, bt * L_REMAIN + j, MASK_ROW)
            pltpu.async_copy(table_hbm.at[cidx0.at[pl.ds(0, 16)]],
                             ga0, gsem0).wait()
            pltpu.async_copy(pos_hbm.at[pl.ds(N, IW), :], pos_v, psem).wait()
            for kk in range(D // 16):
                sl = pl.ds(kk * 16, 16)
                wbuf[0, sl] = ga0[0, sl] + pos_v[0, sl]
            pltpu.sync_copy(wbuf.at[pl.ds(0, 1), :],
                            out_hbm.at[bt, pl.ds(N, 1), :])

    return k(table, ridx_flat, pos_enc, pm_flat)


def kernel(temporal_data, revert_idx, temporal_pos_enc, remain_padding_mask, mask_token):
    table = jnp.concatenate(
        [temporal_data.reshape(B * L_REMAIN, D), mask_token], axis=0)
    ridx_flat = revert_idx.reshape(B * N)
    pm_flat = jnp.pad(remain_padding_mask, ((0, 0), (0, 1))).reshape(B * L_REMAIN)
    return _sc_revert(table, ridx_flat, temporal_pos_enc, pm_flat)


# speedup vs baseline: 1.8185x; 1.0168x over previous
import functools

import jax
import jax.numpy as jnp
from jax import lax
from jax.experimental import pallas as pl
from jax.experimental.pallas import tpu as pltpu
from jax.experimental.pallas import tpu_sc as plsc

B = 4
L_REMAIN = 2048
D = 1024
N = 8192
LFULL = N + 1
MASK_ROW = B * L_REMAIN
IW = 16
WR = B * IW
SQ = N // IW
NC, NS = 2, 16
NW = NC * NS
WPT = SQ // NW


def _sc_revert(table, ridx_flat, pos_enc, pm_flat):
    mesh = plsc.VectorSubcoreMesh(core_axis_name="c", subcore_axis_name="s")

    @functools.partial(
        pl.kernel,
        out_type=jax.ShapeDtypeStruct((B, LFULL, D), jnp.float32),
        mesh=mesh,
        compiler_params=pltpu.CompilerParams(needs_layout_passes=False),
        scratch_types=[
            pltpu.VMEM((B * L_REMAIN,), jnp.int32),
            pltpu.VMEM((B * 24,), jnp.int32),
            pltpu.VMEM((80,), jnp.int32),
            pltpu.VMEM((80,), jnp.int32),
            pltpu.VMEM((80,), jnp.int32),
            pltpu.VMEM((80,), jnp.int32),
            pltpu.VMEM((WR,), jnp.int32),
            pltpu.VMEM((WR,), jnp.int32),
            pltpu.VMEM((16, D), jnp.float32),
            pltpu.VMEM((16, D), jnp.float32),
            pltpu.VMEM((IW, D), jnp.float32),
            pltpu.VMEM((WR, D), jnp.float32),
            pltpu.VMEM((D,), jnp.float32),
            pltpu.SemaphoreType.DMA,
            pltpu.SemaphoreType.DMA,
            pltpu.SemaphoreType.DMA,
            pltpu.SemaphoreType.DMA,
            pltpu.SemaphoreType.DMA,
        ],
    )
    def k(table_hbm, ridx_hbm, pos_hbm, pm_hbm, out_hbm,
          pm_v, rscr, cidx0, cidx1, cdst0, cdst1, val0, val1, ga0, ga1,
          pos_v, wbuf, mt_v, rsem, gsem0, gsem1, psem, wsem):
        cidx_s = (cidx0, cidx1)
        cdst_s = (cdst0, cdst1)
        val_s = (val0, val1)
        ga_s = (ga0, ga1)
        gsem_s = (gsem0, gsem1)

        wid = lax.axis_index("s") * NC + lax.axis_index("c")
        pltpu.sync_copy(pm_hbm, pm_v)
        lanes = lax.iota(jnp.int32, 16)
        cidx0[pl.ds(0, 16)] = jnp.full((16,), MASK_ROW, jnp.int32)
        pltpu.async_copy(table_hbm.at[cidx0.at[pl.ds(0, 16)]], ga0, gsem0).wait()
        for kk in range(D // 16):
            mt_v[pl.ds(kk * 16, 16)] = ga0[0, pl.ds(kk * 16, 16)]

        def win_i0(n):
            return pl.multiple_of((wid * WPT + n) * IW, IW)

        def issue_pos(n):
            pltpu.async_copy(pos_hbm.at[pl.ds(win_i0(n), IW), :], pos_v, psem)

        def wait_pos(n):
            pltpu.make_async_copy(pos_hbm.at[pl.ds(win_i0(n), IW), :], pos_v,
                                  psem).wait()

        def start(n, s):
            i0 = win_i0(n)
            a0 = pl.multiple_of(jnp.maximum(i0 - 8, 0), 8)
            d0 = i0 - a0
            for b in range(B):
                pltpu.async_copy(ridx_hbm.at[pl.ds(b * N + a0, 24)],
                                 rscr.at[pl.ds(24 * b, 24)], rsem)
            for b in range(B):
                pltpu.make_async_copy(ridx_hbm.at[pl.ds(b * N + a0, 24)],
                                      rscr.at[pl.ds(24 * b, 24)], rsem).wait()
            ivec = i0 + lanes
            srcs, masks, counts = [], [], []
            for b in range(B):
                scr_idx = b * 24 + jnp.clip(lanes + d0 - 1, 0, 23)
                r = plsc.load_gather(rscr, [scr_idx])
                j = r + 1
                in_rng = (ivec > 0) & (j <= L_REMAIN - 1)
                fp = jnp.clip(b * L_REMAIN + j - 1, 0, B * L_REMAIN - 1)
                pmv = plsc.load_gather(pm_v, [fp])
                mb = in_rng & (pmv == 1)
                srcs.append(jnp.where(mb, b * L_REMAIN + j, MASK_ROW))
                masks.append(mb)
                counts.append(jnp.sum(mb.astype(jnp.int32)))
            o = jnp.int32(0)
            for b in range(B):
                plsc.store_compressed(cidx_s[s].at[pl.ds(o, 16)], srcs[b],
                                      mask=masks[b])
                plsc.store_compressed(cdst_s[s].at[pl.ds(o, 16)],
                                      16 * b + lanes, mask=masks[b])
                val_s[s][pl.ds(16 * b, 16)] = masks[b].astype(jnp.int32)
                o = o + counts[b]
            v = o
            cidx_s[s][pl.ds(v, 16)] = jnp.full((16,), MASK_ROW, jnp.int32)
            cdst_s[s][pl.ds(v, 16)] = jnp.full((16,), WR - 1, jnp.int32)
            pltpu.async_copy(table_hbm.at[cidx_s[s].at[pl.ds(0, 16)]],
                             ga_s[s], gsem_s[s])
            return v

        def merge_group(s, grp, v):
            def mb(k2, carry2):
                e = grp * 16 + k2
                ev = jnp.full((16,), 1, jnp.int32) * e
                live = ev < v
                dstb = plsc.load_gather(cdst_s[s],
                                        [jnp.full((16,), e, jnp.int32)])
                dstb = jnp.minimum(dstb, WR - 1)
                for kk in range(D // 16):
                    col = kk * 16 + lanes
                    plsc.store_scatter(wbuf, [dstb, col],
                                       ga_s[s][k2, pl.ds(kk * 16, 16)],
                                       mask=live)
                return carry2
            lax.fori_loop(0, 16, mb, 0)

        def drain_writes(n):
            i0p = win_i0(n)
            for b in range(B):
                pltpu.make_async_copy(wbuf.at[pl.ds(b * IW, IW), :],
                                      out_hbm.at[b, pl.ds(i0p, IW), :],
                                      wsem).wait()

        def finish(n, s, v):
            i0 = win_i0(n)

            @pl.when(n >= 1)
            def _():
                drain_writes(n - 1)

            pltpu.make_async_copy(table_hbm.at[cidx_s[s].at[pl.ds(0, 16)]],
                                  ga_s[s], gsem_s[s]).wait()
            merge_group(s, 0, v)
            for grp in range(1, 5):
                @pl.when(v > 16 * grp)
                def _(grp=grp):
                    pltpu.async_copy(
                        table_hbm.at[cidx_s[s].at[pl.ds(16 * grp, 16)]],
                        ga_s[s], gsem_s[s]).wait()
                    merge_group(s, grp, v)
            wait_pos(n)

            def ab(r, carry2):
                il = r - (r // IW) * IW
                vb = plsc.load_gather(val_s[s], [jnp.full((16,), r, jnp.int32)])
                for kk in range(D // 16):
                    sl = pl.ds(kk * 16, 16)
                    w = wbuf[r, sl]
                    wbuf[r, sl] = (
                        jnp.where(vb != 0, w, mt_v[sl]) + pos_v[il, sl])
                return carry2

            lax.fori_loop(0, WR, ab, 0)
            for b in range(B):
                pltpu.async_copy(
                    wbuf.at[pl.ds(b * IW, IW), :],
                    out_hbm.at[b, pl.ds(i0, IW), :], wsem)

            @pl.when(n + 1 < WPT)
            def _():
                issue_pos(n + 1)

        issue_pos(0)
        va = start(0, 0)
        vb = start(1, 1)

        def body(g, carry):
            va, vb = carry
            finish(2 * g, 0, va)
            va2 = start(2 * g + 2, 0)
            finish(2 * g + 1, 1, vb)
            vb2 = start(2 * g + 3, 1)
            return (va2, vb2)

        va, vb = lax.fori_loop(0, WPT // 2 - 1, body, (va, vb))
        finish(WPT - 2, 0, va)
        finish(WPT - 1, 1, vb)
        drain_writes(WPT - 1)

        @pl.when(wid < B)
        def _():
            bt = wid
            pltpu.sync_copy(ridx_hbm.at[pl.ds(bt * N + N - 16, 16)],
                            rscr.at[pl.ds(0, 16)])
            r = plsc.load_gather(rscr, [jnp.full((16,), 15, jnp.int32)])
            j = r + 1
            in_rng = j <= L_REMAIN - 1
            fp = jnp.clip(bt * L_REMAIN + j - 1, 0, B * L_REMAIN - 1)
            pmv = plsc.load_gather(pm_v, [fp])
            valid = in_rng & (pmv == 1)
            cidx0[pl.ds(0, 16)] = jnp.where(valid, bt * L_REMAIN + j, MASK_ROW)
            pltpu.async_copy(table_hbm.at[cidx0.at[pl.ds(0, 16)]],
                             ga0, gsem0).wait()
            pltpu.async_copy(pos_hbm.at[pl.ds(N, IW), :], pos_v, psem).wait()
            for kk in range(D // 16):
                sl = pl.ds(kk * 16, 16)
                wbuf[0, sl] = ga0[0, sl] + pos_v[0, sl]
            pltpu.sync_copy(wbuf.at[pl.ds(0, 1), :],
                            out_hbm.at[bt, pl.ds(N, 1), :])

    return k(table, ridx_flat, pos_enc, pm_flat)


def kernel(temporal_data, revert_idx, temporal_pos_enc, remain_padding_mask, mask_token):
    table = jnp.concatenate(
        [temporal_data.reshape(B * L_REMAIN, D), mask_token], axis=0)
    ridx_flat = revert_idx.reshape(B * N)
    pm_flat = jnp.pad(remain_padding_mask, ((0, 0), (0, 1))).reshape(B * L_REMAIN)
    return _sc_revert(table, ridx_flat, temporal_pos_enc, pm_flat)
